# baseline (speedup 1.0000x reference)
import functools

import jax
import jax.numpy as jnp
from jax import lax
from jax.experimental import pallas as pl
from jax.experimental.pallas import tpu as pltpu
from jax.experimental.pallas import tpu_sc as plsc

B, N, KPOOL, D, C, R, KR = 64, 50000, 5000, 512, 25, 8, 4
NEG = -1e9
NPAD = 51200
KPAD = 5120
GW = 2560
KWIN = 384
MCH = 2000
BIGI = 2**30


def _sc_gather_translate(table, order, pos):
    Bn = pos.shape[0]
    Dt = table.shape[1]
    On = order.shape[0]
    chunk = 80
    info = plsc.get_sparse_core_info()
    NW = info.num_cores * info.num_subcores
    per_w = Bn // NW
    nchunks = per_w // chunk
    mesh = plsc.VectorSubcoreMesh(core_axis_name="c", subcore_axis_name="s")

    @functools.partial(
        pl.kernel,
        mesh=mesh,
        out_type=jax.ShapeDtypeStruct((Bn, Dt), jnp.float32),
        compiler_params=pltpu.CompilerParams(needs_layout_passes=False),
        scratch_types=[
            pltpu.VMEM((On,), jnp.int32),
            pltpu.VMEM((chunk,), jnp.int32),
            pltpu.VMEM((chunk,), jnp.int32),
            pltpu.VMEM((chunk, Dt), jnp.float32),
            pltpu.SemaphoreType.DMA,
        ],
    )
    def k(table_hbm, order_hbm, pos_hbm, out_hbm, order_v, pos_v, idx_v,
          rows_v, sem):
        wid = lax.axis_index("s") * info.num_cores + lax.axis_index("c")
        pltpu.sync_copy(order_hbm, order_v)

        def body(j, _):
            base = wid * per_w + j * chunk
            pltpu.sync_copy(pos_hbm.at[pl.ds(base, chunk)], pos_v)
            for g in range(chunk // 16):
                pv = pos_v[pl.ds(g * 16, 16)]
                idx_v[pl.ds(g * 16, 16)] = plsc.load_gather(order_v, [pv])
            pltpu.async_copy(table_hbm.at[idx_v], rows_v, sem).wait()
            pltpu.sync_copy(rows_v, out_hbm.at[pl.ds(base, chunk)])
            return 0

        lax.fori_loop(0, nchunks, body, 0)

    return k(table, order, pos)


def _sc_gather_simple(table, idx, chunk, c0_frac_32nds=16):
    Bn = idx.shape[0]
    Dt = table.shape[1]
    info = plsc.get_sparse_core_info()
    NS = info.num_subcores
    w0 = (Bn * c0_frac_32nds // 32) // (NS * chunk) * chunk
    w1 = (Bn - w0 * NS) // NS
    assert w1 % chunk == 0 and (w0 + w1) * NS == Bn
    n0 = w0 // chunk
    n1 = w1 // chunk
    mesh = plsc.VectorSubcoreMesh(core_axis_name="c", subcore_axis_name="s")

    @functools.partial(
        pl.kernel,
        mesh=mesh,
        out_type=jax.ShapeDtypeStruct((Bn, Dt), jnp.float32),
        compiler_params=pltpu.CompilerParams(use_tc_tiling_on_sc=False),
        scratch_types=[
            pltpu.VMEM((chunk,), jnp.int32),
            pltpu.VMEM((chunk, Dt), jnp.float32),
            pltpu.SemaphoreType.DMA,
        ],
    )
    def k(table_hbm, idx_hbm, out_hbm, idx_v, rows_v, sem):
        c = lax.axis_index("c")
        s = lax.axis_index("s")
        wbase = jnp.where(c == 0, s * w0, NS * w0 + s * w1)
        nchunks = jnp.where(c == 0, n0, n1)

        def body(j, _):
            base = wbase + j * chunk
            pltpu.sync_copy(idx_hbm.at[pl.ds(base, chunk)], idx_v)
            pltpu.async_copy(table_hbm.at[idx_v], rows_v, sem).wait()
            pltpu.sync_copy(rows_v, out_hbm.at[pl.ds(base, chunk)])
            return 0

        lax.fori_loop(0, nchunks, body, 0)

    return k(table, idx)


def _sc_gather(table, idx, chunk):
    Bn = idx.shape[0]
    Dt = table.shape[1]
    info = plsc.get_sparse_core_info()
    NW = info.num_cores * info.num_subcores
    per_w = Bn // NW
    nc = per_w // chunk
    mesh = plsc.VectorSubcoreMesh(core_axis_name="c", subcore_axis_name="s")

    @functools.partial(
        pl.kernel,
        mesh=mesh,
        out_type=jax.ShapeDtypeStruct((Bn, Dt), jnp.float32),
        scratch_types=[
            pltpu.VMEM((per_w,), jnp.int32),
            pltpu.VMEM((chunk, Dt), jnp.float32),
            pltpu.VMEM((chunk, Dt), jnp.float32),
            pltpu.SemaphoreType.DMA,
            pltpu.SemaphoreType.DMA,
            pltpu.SemaphoreType.DMA,
            pltpu.SemaphoreType.DMA,
        ],
    )
    def k(table_hbm, idx_hbm, out_hbm, idx_all, buf0, buf1,
          g0, g1, s0, s1):
        wid = lax.axis_index("s") * info.num_cores + lax.axis_index("c")
        wbase = wid * per_w
        pltpu.sync_copy(idx_hbm.at[pl.ds(wbase, per_w)], idx_all)
        bufs = (buf0, buf1)
        gsems = (g0, g1)
        ssems = (s0, s1)
        gcs = [None] * nc
        sts = [None] * nc
        for j in range(nc):
            b = j & 1
            if j >= 2:
                sts[j - 2].wait()
            gcs[j] = pltpu.async_copy(
                table_hbm.at[idx_all.at[pl.ds(j * chunk, chunk)]],
                bufs[b], gsems[b])
            if j >= 1:
                gcs[j - 1].wait()
                sts[j - 1] = pltpu.async_copy(
                    bufs[(j - 1) & 1],
                    out_hbm.at[pl.ds(wbase + (j - 1) * chunk, chunk)],
                    ssems[(j - 1) & 1])
        gcs[nc - 1].wait()
        sts[nc - 1] = pltpu.async_copy(
            bufs[(nc - 1) & 1],
            out_hbm.at[pl.ds(wbase + (nc - 1) * chunk, chunk)],
            ssems[(nc - 1) & 1])
        if nc >= 2:
            sts[nc - 2].wait()
        sts[nc - 1].wait()

    return k(table, idx)


def _t0_body(g_ref, q_ref, out_ref):
    out_ref[...] = lax.dot_general(g_ref[...], q_ref[...], (((1,), (1,)), ((), ())),
                                   preferred_element_type=jnp.float32)


def _t0_sims(g, q):
    return pl.pallas_call(
        _t0_body,
        grid=(N // MCH,),
        in_specs=[
            pl.BlockSpec((MCH, D), lambda n: (n, 0)),
            pl.BlockSpec((B, D), lambda n: (0, 0)),
        ],
        out_specs=pl.BlockSpec((MCH, B), lambda n: (n, 0)),
        out_shape=jax.ShapeDtypeStruct((N, B), jnp.float32),
    )(g, q)


T1TILE = 128
NT1 = GW // T1TILE


def _t1_body(starts_ref, ends_ref, sims_hbm, att_ref, pos_ref, win_ref, sem):
    c = pl.program_id(0)
    s = starts_ref[c]
    e = ends_ref[c]
    base = jnp.minimum((s // 8) * 8, NPAD - GW)
    cp = pltpu.make_async_copy(sims_hbm.at[pl.ds(base, GW)], win_ref, sem)
    cp.start()
    cp.wait()
    lo = s - base
    hi = e - base
    tio = lax.broadcasted_iota(jnp.int32, (T1TILE, B), 0)
    cv = []
    ci = []
    for t in range(NT1):
        St = win_ref[pl.ds(t * T1TILE, T1TILE), :]
        rio = tio + t * T1TILE
        cur = jnp.where((rio >= lo) & (rio < hi), St, NEG)
        for _ in range(R):
            m = jnp.max(cur, axis=0, keepdims=True)
            idx = jnp.min(jnp.where(cur == m, rio, BIGI), axis=0, keepdims=True)
            cv.append(m)
            ci.append(idx)
            cur = jnp.where(rio == idx, NEG, cur)
    V = jnp.concatenate(cv, axis=0)
    I = jnp.concatenate(ci, axis=0)
    cio = lax.broadcasted_iota(jnp.int32, (NT1 * R, B), 0)
    vals = []
    poss = []
    for _ in range(R):
        m = jnp.max(V, axis=0, keepdims=True)
        pick = jnp.min(jnp.where(V == m, cio, BIGI), axis=0, keepdims=True)
        sel = jnp.sum(jnp.where(cio == pick, I, 0), axis=0, keepdims=True)
        vals.append(m)
        poss.append(sel + base)
        V = jnp.where(cio == pick, NEG, V)
    v8 = jnp.concatenate(vals, axis=0)
    mw = jnp.max(v8, axis=0, keepdims=True)
    ew = jnp.exp(v8 - mw)
    att_ref[...] = (ew / jnp.sum(ew, axis=0, keepdims=True))[None]
    pos_ref[...] = jnp.concatenate(poss, axis=0)[None]


def _t1_topk(starts, ends, sims_sorted):
    return pl.pallas_call(
        _t1_body,
        grid=(C,),
        in_specs=[
            pl.BlockSpec(memory_space=pltpu.MemorySpace.SMEM),
            pl.BlockSpec(memory_space=pltpu.MemorySpace.SMEM),
            pl.BlockSpec(memory_space=pltpu.MemorySpace.HBM),
        ],
        out_specs=[
            pl.BlockSpec((1, R, B), lambda c: (c, 0, 0)),
            pl.BlockSpec((1, R, B), lambda c: (c, 0, 0)),
        ],
        out_shape=[
            jax.ShapeDtypeStruct((C, R, B), jnp.float32),
            jax.ShapeDtypeStruct((C, R, B), jnp.int32),
        ],
        scratch_shapes=[
            pltpu.VMEM((GW, B), jnp.float32),
            pltpu.SemaphoreType.DMA,
        ],
    )(starts, ends, sims_sorted)


def _t2_body(q_ref, x_ref, att_ref, kstarts_ref, kends_ref, khbm_ref,
             out_img_ref, out_know_ref, kseg_ref, sem):
    c = pl.program_id(0)
    ks = kstarts_ref[c]
    ke = kends_ref[c]
    base = jnp.minimum((ks // 8) * 8, KPAD - KWIN)
    cp = pltpu.make_async_copy(khbm_ref.at[pl.ds(base, KWIN)], kseg_ref, sem)
    cp.start()
    cp.wait()
    X = x_ref[0]
    Kseg = kseg_ref[...]
    S2 = lax.dot_general(X, Kseg, (((1,), (1,)), ((), ())),
                         preferred_element_type=jnp.float32)
    T2T = 64
    cio = lax.broadcasted_iota(jnp.int32, (T2T, KWIN), 1) + base
    m1s = []
    t4s = []
    for t in range((R * B) // T2T):
        St = S2[t * T2T:(t + 1) * T2T, :]
        cur = jnp.where((cio >= ks) & (cio < ke), St, NEG)
        m = None
        for r in range(KR):
            m = jnp.max(cur, axis=1, keepdims=True)
            if r == 0:
                m1s.append(m)
            if r < KR - 1:
                cur = jnp.where(cur == m, NEG, cur)
        t4s.append(m)
    m1 = jnp.concatenate(m1s, axis=0)
    t4 = jnp.concatenate(t4s, axis=0)
    iota = lax.broadcasted_iota(jnp.int32, (R * B, KWIN), 1)
    colk = iota + base
    Sm = jnp.where((colk >= ks) & (colk < ke), S2, NEG)
    Wk = jnp.where(Sm >= t4, jnp.exp(Sm - m1), 0.0)
    denom = jnp.sum(Wk, axis=1, keepdims=True)
    A = Wk / denom
    per_img = lax.dot_general(A.astype(jnp.bfloat16), Kseg.astype(jnp.bfloat16),
                              (((1,), (0,)), ((), ())),
                              preferred_element_type=jnp.float32)
    att = att_ref[0]
    att_flat = jnp.concatenate([att[r:r + 1, :] for r in range(R)], axis=1)
    biota = lax.broadcasted_iota(jnp.int32, (B, R * B), 0)
    colmod = lax.broadcasted_iota(jnp.int32, (B, R * B), 1) % B
    W3 = jnp.where(colmod == biota, att_flat, 0.0)
    ctx_img = lax.dot_general(W3, X, (((1,), (0,)), ((), ())),
                              preferred_element_type=jnp.float32)
    ctx_know = lax.dot_general(W3, per_img, (((1,), (0,)), ((), ())),
                               preferred_element_type=jnp.float32)
    q = q_ref[...]
    out_img_ref[...] = (0.5 * q + 0.5 * ctx_img)[None]
    out_know_ref[...] = (0.5 * q + 0.5 * ctx_know)[None]


def _t2_stage2(q, x_img, att, kstarts, kends, k_sorted):
    return pl.pallas_call(
        _t2_body,
        grid=(C,),
        in_specs=[
            pl.BlockSpec((B, D), lambda c: (0, 0)),
            pl.BlockSpec((1, R * B, D), lambda c: (c, 0, 0)),
            pl.BlockSpec((1, R, B), lambda c: (c, 0, 0)),
            pl.BlockSpec(memory_space=pltpu.MemorySpace.SMEM),
            pl.BlockSpec(memory_space=pltpu.MemorySpace.SMEM),
            pl.BlockSpec(memory_space=pltpu.MemorySpace.HBM),
        ],
        out_specs=[
            pl.BlockSpec((1, B, D), lambda c: (c, 0, 0)),
            pl.BlockSpec((1, B, D), lambda c: (c, 0, 0)),
        ],
        out_shape=[
            jax.ShapeDtypeStruct((C, B, D), jnp.float32),
            jax.ShapeDtypeStruct((C, B, D), jnp.float32),
        ],
        scratch_shapes=[
            pltpu.VMEM((KWIN, D), jnp.float32),
            pltpu.SemaphoreType.DMA,
        ],
    )(q, x_img, att, kstarts, kends, k_sorted)


def kernel(query_embeddings, all_image_embeddings, all_knowledge_embeddings,
           image_labels, knowledge_labels):
    classes = jnp.arange(C, dtype=jnp.int32)
    il = image_labels.astype(jnp.int32)
    kltyp = knowledge_labels.astype(jnp.int32)

    ikeys = il * 131072 + jnp.arange(N, dtype=jnp.int32)
    img_order = lax.sort(ikeys) & 131071
    starts = jnp.sum(il[None, :] < classes[:, None], axis=1).astype(jnp.int32)
    ends = jnp.sum(il[None, :] <= classes[:, None], axis=1).astype(jnp.int32)
    img_order_p = jnp.concatenate([img_order, jnp.zeros((NPAD - N,), jnp.int32)])

    kkeys = kltyp * 8192 + jnp.arange(KPOOL, dtype=jnp.int32)
    korder = lax.sort(kkeys) & 8191
    kstarts = jnp.sum(kltyp[None, :] < classes[:, None], axis=1).astype(jnp.int32)
    kends = jnp.sum(kltyp[None, :] <= classes[:, None], axis=1).astype(jnp.int32)
    korder_p = jnp.concatenate([korder, jnp.zeros((KPAD - KPOOL,), jnp.int32)])

    k_sorted = _sc_gather(all_knowledge_embeddings, korder_p, 80)
    sims = _t0_sims(all_image_embeddings, query_embeddings)
    sims_sorted = _sc_gather_simple(sims, img_order_p, 80, 21)
    att, pos = _t1_topk(starts, ends, sims_sorted)
    x_img = _sc_gather_translate(all_image_embeddings, img_order_p,
                                 pos.reshape(-1)).reshape(C, R * B, D)
    out_img, out_know = _t2_stage2(query_embeddings, x_img, att,
                                   kstarts, kends, k_sorted)
    return (jnp.transpose(out_img, (1, 0, 2)),
            jnp.transpose(out_know, (1, 0, 2)))


# speedup vs baseline: 1.1875x; 1.1875x over previous
import functools

import jax
import jax.numpy as jnp
from jax import lax
from jax.experimental import pallas as pl
from jax.experimental.pallas import tpu as pltpu
from jax.experimental.pallas import tpu_sc as plsc

B, N, KPOOL, D, C, R, KR = 64, 50000, 5000, 512, 25, 8, 4
NEG = -1e9
NPAD = 51200
KPAD = 5120
GW = 2560
KWIN = 384
MCH = 2000
BIGI = 2**30


def _sc_gather_translate(table, order, pos):
    Bn = pos.shape[0]
    Dt = table.shape[1]
    On = order.shape[0]
    chunk = 80
    info = plsc.get_sparse_core_info()
    NW = info.num_cores * info.num_subcores
    per_w = Bn // NW
    nchunks = per_w // chunk
    mesh = plsc.VectorSubcoreMesh(core_axis_name="c", subcore_axis_name="s")

    @functools.partial(
        pl.kernel,
        mesh=mesh,
        out_type=jax.ShapeDtypeStruct((Bn, Dt), jnp.float32),
        compiler_params=pltpu.CompilerParams(needs_layout_passes=False),
        scratch_types=[
            pltpu.VMEM((On,), jnp.int32),
            pltpu.VMEM((chunk,), jnp.int32),
            pltpu.VMEM((chunk,), jnp.int32),
            pltpu.VMEM((chunk, Dt), jnp.float32),
            pltpu.SemaphoreType.DMA,
        ],
    )
    def k(table_hbm, order_hbm, pos_hbm, out_hbm, order_v, pos_v, idx_v,
          rows_v, sem):
        wid = lax.axis_index("s") * info.num_cores + lax.axis_index("c")
        pltpu.sync_copy(order_hbm, order_v)

        def body(j, _):
            base = wid * per_w + j * chunk
            pltpu.sync_copy(pos_hbm.at[pl.ds(base, chunk)], pos_v)
            for g in range(chunk // 16):
                pv = pos_v[pl.ds(g * 16, 16)]
                idx_v[pl.ds(g * 16, 16)] = plsc.load_gather(order_v, [pv])
            pltpu.async_copy(table_hbm.at[idx_v], rows_v, sem).wait()
            pltpu.sync_copy(rows_v, out_hbm.at[pl.ds(base, chunk)])
            return 0

        lax.fori_loop(0, nchunks, body, 0)

    return k(table, order, pos)


def _sc_gather_simple(table, idx, chunk, c0_frac_32nds=16):
    Bn = idx.shape[0]
    Dt = table.shape[1]
    info = plsc.get_sparse_core_info()
    NS = info.num_subcores
    w0 = (Bn * c0_frac_32nds // 32) // (NS * chunk) * chunk
    w1 = (Bn - w0 * NS) // NS
    assert w1 % chunk == 0 and (w0 + w1) * NS == Bn
    n0 = w0 // chunk
    n1 = w1 // chunk
    mesh = plsc.VectorSubcoreMesh(core_axis_name="c", subcore_axis_name="s")

    @functools.partial(
        pl.kernel,
        mesh=mesh,
        out_type=jax.ShapeDtypeStruct((Bn, Dt), jnp.float32),
        scratch_types=[
            pltpu.VMEM((chunk,), jnp.int32),
            pltpu.VMEM((chunk, Dt), jnp.float32),
            pltpu.SemaphoreType.DMA,
        ],
    )
    def k(table_hbm, idx_hbm, out_hbm, idx_v, rows_v, sem):
        c = lax.axis_index("c")
        s = lax.axis_index("s")
        wbase = jnp.where(c == 0, s * w0, NS * w0 + s * w1)
        nchunks = jnp.where(c == 0, n0, n1)

        def body(j, _):
            base = wbase + j * chunk
            pltpu.sync_copy(idx_hbm.at[pl.ds(base, chunk)], idx_v)
            pltpu.async_copy(table_hbm.at[idx_v], rows_v, sem).wait()
            pltpu.sync_copy(rows_v, out_hbm.at[pl.ds(base, chunk)])
            return 0

        lax.fori_loop(0, nchunks, body, 0)

    return k(table, idx)


def _sc_gather(table, idx, chunk):
    Bn = idx.shape[0]
    Dt = table.shape[1]
    info = plsc.get_sparse_core_info()
    NW = info.num_cores * info.num_subcores
    per_w = Bn // NW
    nc = per_w // chunk
    mesh = plsc.VectorSubcoreMesh(core_axis_name="c", subcore_axis_name="s")

    @functools.partial(
        pl.kernel,
        mesh=mesh,
        out_type=jax.ShapeDtypeStruct((Bn, Dt), jnp.float32),
        scratch_types=[
            pltpu.VMEM((per_w,), jnp.int32),
            pltpu.VMEM((chunk, Dt), jnp.float32),
            pltpu.VMEM((chunk, Dt), jnp.float32),
            pltpu.SemaphoreType.DMA,
            pltpu.SemaphoreType.DMA,
            pltpu.SemaphoreType.DMA,
            pltpu.SemaphoreType.DMA,
        ],
    )
    def k(table_hbm, idx_hbm, out_hbm, idx_all, buf0, buf1,
          g0, g1, s0, s1):
        wid = lax.axis_index("s") * info.num_cores + lax.axis_index("c")
        wbase = wid * per_w
        pltpu.sync_copy(idx_hbm.at[pl.ds(wbase, per_w)], idx_all)
        bufs = (buf0, buf1)
        gsems = (g0, g1)
        ssems = (s0, s1)
        gcs = [None] * nc
        sts = [None] * nc
        for j in range(nc):
            b = j & 1
            if j >= 2:
                sts[j - 2].wait()
            gcs[j] = pltpu.async_copy(
                table_hbm.at[idx_all.at[pl.ds(j * chunk, chunk)]],
                bufs[b], gsems[b])
            if j >= 1:
                gcs[j - 1].wait()
                sts[j - 1] = pltpu.async_copy(
                    bufs[(j - 1) & 1],
                    out_hbm.at[pl.ds(wbase + (j - 1) * chunk, chunk)],
                    ssems[(j - 1) & 1])
        gcs[nc - 1].wait()
        sts[nc - 1] = pltpu.async_copy(
            bufs[(nc - 1) & 1],
            out_hbm.at[pl.ds(wbase + (nc - 1) * chunk, chunk)],
            ssems[(nc - 1) & 1])
        if nc >= 2:
            sts[nc - 2].wait()
        sts[nc - 1].wait()

    return k(table, idx)


def _t0_body(g_ref, q_ref, out_ref):
    out_ref[...] = lax.dot_general(g_ref[...], q_ref[...], (((1,), (1,)), ((), ())),
                                   preferred_element_type=jnp.float32)


def _t0_sims(g, q_pad):
    return pl.pallas_call(
        _t0_body,
        grid=(N // MCH,),
        in_specs=[
            pl.BlockSpec((MCH, D), lambda n: (n, 0)),
            pl.BlockSpec((2 * B, D), lambda n: (0, 0)),
        ],
        out_specs=pl.BlockSpec((MCH, 2 * B), lambda n: (n, 0)),
        out_shape=jax.ShapeDtypeStruct((N, 2 * B), jnp.float32),
    )(g, q_pad)


T1TILE = 128
NT1 = GW // T1TILE
CP = C + 1


def _t1_body(starts_ref, ends_ref, sims_hbm, att_ref, pos_ref,
             win0_ref, win1_ref, sem0, sem1):
    p = pl.program_id(0)
    s0 = starts_ref[2 * p]
    e0 = ends_ref[2 * p]
    s1 = starts_ref[2 * p + 1]
    e1 = ends_ref[2 * p + 1]
    base0 = jnp.minimum((s0 // 8) * 8, NPAD - GW)
    base1 = jnp.minimum((s1 // 8) * 8, NPAD - GW)
    cp0 = pltpu.make_async_copy(sims_hbm.at[pl.ds(base0, GW)], win0_ref, sem0)
    cp1 = pltpu.make_async_copy(sims_hbm.at[pl.ds(base1, GW)], win1_ref, sem1)
    cp0.start()
    cp1.start()
    cp0.wait()
    cp1.wait()
    lane = lax.broadcasted_iota(jnp.int32, (1, 2 * B), 1)
    left = lane < B
    lo_v = jnp.where(left, s0 - base0, s1 - base1)
    hi_v = jnp.where(left, e0 - base0, e1 - base1)
    base_v = jnp.where(left, base0, base1)
    tio = lax.broadcasted_iota(jnp.int32, (T1TILE, 2 * B), 0)
    lmask = lax.broadcasted_iota(jnp.int32, (T1TILE, 2 * B), 1) < B
    cv = []
    ci = []
    for t in range(NT1):
        St0 = win0_ref[pl.ds(t * T1TILE, T1TILE), :]
        St1 = win1_ref[pl.ds(t * T1TILE, T1TILE), :]
        St = jnp.where(lmask, St0, St1)
        rio = tio + t * T1TILE
        cur = jnp.where((rio >= lo_v) & (rio < hi_v), St, NEG)
        for _ in range(R):
            m = jnp.max(cur, axis=0, keepdims=True)
            idx = jnp.min(jnp.where(cur == m, rio, BIGI), axis=0, keepdims=True)
            cv.append(m)
            ci.append(idx)
            cur = jnp.where(rio == idx, NEG, cur)
    V = jnp.concatenate(cv, axis=0)
    I = jnp.concatenate(ci, axis=0)
    cio = lax.broadcasted_iota(jnp.int32, (NT1 * R, 2 * B), 0)
    vals = []
    poss = []
    for _ in range(R):
        m = jnp.max(V, axis=0, keepdims=True)
        pick = jnp.min(jnp.where(V == m, cio, BIGI), axis=0, keepdims=True)
        sel = jnp.sum(jnp.where(cio == pick, I, 0), axis=0, keepdims=True)
        vals.append(m)
        poss.append(sel + base_v)
        V = jnp.where(cio == pick, NEG, V)
    v8 = jnp.concatenate(vals, axis=0)
    mw = jnp.max(v8, axis=0, keepdims=True)
    ew = jnp.exp(v8 - mw)
    att = ew / jnp.sum(ew, axis=0, keepdims=True)
    pos = jnp.concatenate(poss, axis=0)
    att_ref[...] = jnp.concatenate(
        [att[:, :B][None], att[:, B:][None]], axis=0)
    pos_ref[...] = jnp.concatenate(
        [pos[:, :B][None], pos[:, B:][None]], axis=0)


def _t1_topk(starts, ends, sims_sorted):
    return pl.pallas_call(
        _t1_body,
        grid=(CP // 2,),
        in_specs=[
            pl.BlockSpec(memory_space=pltpu.MemorySpace.SMEM),
            pl.BlockSpec(memory_space=pltpu.MemorySpace.SMEM),
            pl.BlockSpec(memory_space=pltpu.MemorySpace.HBM),
        ],
        out_specs=[
            pl.BlockSpec((2, R, B), lambda p: (p, 0, 0)),
            pl.BlockSpec((2, R, B), lambda p: (p, 0, 0)),
        ],
        out_shape=[
            jax.ShapeDtypeStruct((CP, R, B), jnp.float32),
            jax.ShapeDtypeStruct((CP, R, B), jnp.int32),
        ],
        scratch_shapes=[
            pltpu.VMEM((GW, 2 * B), jnp.float32),
            pltpu.VMEM((GW, 2 * B), jnp.float32),
            pltpu.SemaphoreType.DMA,
            pltpu.SemaphoreType.DMA,
        ],
    )(starts, ends, sims_sorted)


def _t2_body(q_ref, x_ref, att_ref, kstarts_ref, kends_ref, khbm_ref,
             out_img_ref, out_know_ref, kseg_ref, sem):
    c = pl.program_id(0)
    ks = kstarts_ref[c]
    ke = kends_ref[c]
    base = jnp.minimum((ks // 8) * 8, KPAD - KWIN)
    cp = pltpu.make_async_copy(khbm_ref.at[pl.ds(base, KWIN)], kseg_ref, sem)
    cp.start()
    cp.wait()
    X = x_ref[0]
    Kseg = kseg_ref[...]
    S2 = lax.dot_general(X, Kseg, (((1,), (1,)), ((), ())),
                         preferred_element_type=jnp.float32)
    T2T = 64
    cio = lax.broadcasted_iota(jnp.int32, (T2T, KWIN), 1) + base
    m1s = []
    t4s = []
    for t in range((R * B) // T2T):
        St = S2[t * T2T:(t + 1) * T2T, :]
        cur = jnp.where((cio >= ks) & (cio < ke), St, NEG)
        m = None
        for r in range(KR):
            m = jnp.max(cur, axis=1, keepdims=True)
            if r == 0:
                m1s.append(m)
            if r < KR - 1:
                cur = jnp.where(cur == m, NEG, cur)
        t4s.append(m)
    m1 = jnp.concatenate(m1s, axis=0)
    t4 = jnp.concatenate(t4s, axis=0)
    iota = lax.broadcasted_iota(jnp.int32, (R * B, KWIN), 1)
    colk = iota + base
    Sm = jnp.where((colk >= ks) & (colk < ke), S2, NEG)
    Wk = jnp.where(Sm >= t4, jnp.exp(Sm - m1), 0.0)
    denom = jnp.sum(Wk, axis=1, keepdims=True)
    A = Wk / denom
    per_img = lax.dot_general(A.astype(jnp.bfloat16), Kseg.astype(jnp.bfloat16),
                              (((1,), (0,)), ((), ())),
                              preferred_element_type=jnp.float32)
    att = att_ref[0]
    att_flat = jnp.concatenate([att[r:r + 1, :] for r in range(R)], axis=1)
    biota = lax.broadcasted_iota(jnp.int32, (B, R * B), 0)
    colmod = lax.broadcasted_iota(jnp.int32, (B, R * B), 1) % B
    W3 = jnp.where(colmod == biota, att_flat, 0.0)
    ctx_img = lax.dot_general(W3, X, (((1,), (0,)), ((), ())),
                              preferred_element_type=jnp.float32)
    ctx_know = lax.dot_general(W3, per_img, (((1,), (0,)), ((), ())),
                               preferred_element_type=jnp.float32)
    q = q_ref[...]
    out_img_ref[...] = (0.5 * q + 0.5 * ctx_img)[None]
    out_know_ref[...] = (0.5 * q + 0.5 * ctx_know)[None]


def _t2_stage2(q, x_img, att, kstarts, kends, k_sorted):
    return pl.pallas_call(
        _t2_body,
        grid=(C,),
        in_specs=[
            pl.BlockSpec((B, D), lambda c: (0, 0)),
            pl.BlockSpec((1, R * B, D), lambda c: (c, 0, 0)),
            pl.BlockSpec((1, R, B), lambda c: (c, 0, 0)),
            pl.BlockSpec(memory_space=pltpu.MemorySpace.SMEM),
            pl.BlockSpec(memory_space=pltpu.MemorySpace.SMEM),
            pl.BlockSpec(memory_space=pltpu.MemorySpace.HBM),
        ],
        out_specs=[
            pl.BlockSpec((1, B, D), lambda c: (c, 0, 0)),
            pl.BlockSpec((1, B, D), lambda c: (c, 0, 0)),
        ],
        out_shape=[
            jax.ShapeDtypeStruct((C, B, D), jnp.float32),
            jax.ShapeDtypeStruct((C, B, D), jnp.float32),
        ],
        scratch_shapes=[
            pltpu.VMEM((KWIN, D), jnp.float32),
            pltpu.SemaphoreType.DMA,
        ],
    )(q, x_img, att, kstarts, kends, k_sorted)


def kernel(query_embeddings, all_image_embeddings, all_knowledge_embeddings,
           image_labels, knowledge_labels):
    classes = jnp.arange(C, dtype=jnp.int32)
    il = image_labels.astype(jnp.int32)
    kltyp = knowledge_labels.astype(jnp.int32)

    ikeys = il * 131072 + jnp.arange(N, dtype=jnp.int32)
    img_order = lax.sort(ikeys) & 131071
    starts = jnp.sum(il[None, :] < classes[:, None], axis=1).astype(jnp.int32)
    ends = jnp.sum(il[None, :] <= classes[:, None], axis=1).astype(jnp.int32)
    img_order_p = jnp.concatenate([img_order, jnp.zeros((NPAD - N,), jnp.int32)])

    kkeys = kltyp * 8192 + jnp.arange(KPOOL, dtype=jnp.int32)
    korder = lax.sort(kkeys) & 8191
    kstarts = jnp.sum(kltyp[None, :] < classes[:, None], axis=1).astype(jnp.int32)
    kends = jnp.sum(kltyp[None, :] <= classes[:, None], axis=1).astype(jnp.int32)
    korder_p = jnp.concatenate([korder, jnp.zeros((KPAD - KPOOL,), jnp.int32)])

    k_sorted = _sc_gather(all_knowledge_embeddings, korder_p, 80)
    q_pad = jnp.concatenate([query_embeddings, query_embeddings], axis=0)
    sims = _t0_sims(all_image_embeddings, q_pad)
    sims_sorted = _sc_gather_simple(sims, img_order_p, 80, 21)
    starts_p = jnp.concatenate([starts, jnp.full((CP - C,), N, jnp.int32)])
    ends_p = jnp.concatenate([ends, jnp.full((CP - C,), N, jnp.int32)])
    att, pos = _t1_topk(starts_p, ends_p, sims_sorted)
    x_img = _sc_gather_translate(all_image_embeddings, img_order_p,
                                 pos[:C].reshape(-1)).reshape(C, R * B, D)
    out_img, out_know = _t2_stage2(query_embeddings, x_img, att,
                                   kstarts, kends, k_sorted)
    return (jnp.transpose(out_img, (1, 0, 2)),
            jnp.transpose(out_know, (1, 0, 2)))


# speedup vs baseline: 1.3078x; 1.1013x over previous
import functools

import jax
import jax.numpy as jnp
from jax import lax
from jax.experimental import pallas as pl
from jax.experimental.pallas import tpu as pltpu
from jax.experimental.pallas import tpu_sc as plsc

B, N, KPOOL, D, C, R, KR = 64, 50000, 5000, 512, 25, 8, 4
NEG = -1e9
NPAD = 51200
KPAD = 5120
GW = 2560
KWIN = 384
MCH = 2000
BIGI = 2**30


def _sc_gather_translate(table, order, pos):
    Bn = pos.shape[0]
    Dt = table.shape[1]
    On = order.shape[0]
    chunk = 80
    info = plsc.get_sparse_core_info()
    NW = info.num_cores * info.num_subcores
    per_w = Bn // NW
    nchunks = per_w // chunk
    mesh = plsc.VectorSubcoreMesh(core_axis_name="c", subcore_axis_name="s")

    @functools.partial(
        pl.kernel,
        mesh=mesh,
        out_type=jax.ShapeDtypeStruct((Bn, Dt), jnp.float32),
        compiler_params=pltpu.CompilerParams(needs_layout_passes=False),
        scratch_types=[
            pltpu.VMEM((On,), jnp.int32),
            pltpu.VMEM((chunk,), jnp.int32),
            pltpu.VMEM((chunk,), jnp.int32),
            pltpu.VMEM((chunk, Dt), jnp.float32),
            pltpu.SemaphoreType.DMA,
        ],
    )
    def k(table_hbm, order_hbm, pos_hbm, out_hbm, order_v, pos_v, idx_v,
          rows_v, sem):
        wid = lax.axis_index("s") * info.num_cores + lax.axis_index("c")
        pltpu.sync_copy(order_hbm, order_v)

        def body(j, _):
            base = wid * per_w + j * chunk
            pltpu.sync_copy(pos_hbm.at[pl.ds(base, chunk)], pos_v)
            for g in range(chunk // 16):
                pv = pos_v[pl.ds(g * 16, 16)]
                idx_v[pl.ds(g * 16, 16)] = plsc.load_gather(order_v, [pv])
            pltpu.async_copy(table_hbm.at[idx_v], rows_v, sem).wait()
            pltpu.sync_copy(rows_v, out_hbm.at[pl.ds(base, chunk)])
            return 0

        lax.fori_loop(0, nchunks, body, 0)

    return k(table, order, pos)


def _sc_gather_simple(table, idx, chunk, c0_frac_32nds=16):
    Bn = idx.shape[0]
    Dt = table.shape[1]
    info = plsc.get_sparse_core_info()
    NS = info.num_subcores
    w0 = (Bn * c0_frac_32nds // 32) // (NS * chunk) * chunk
    w1 = (Bn - w0 * NS) // NS
    assert w1 % chunk == 0 and (w0 + w1) * NS == Bn
    n0 = w0 // chunk
    n1 = w1 // chunk
    mesh = plsc.VectorSubcoreMesh(core_axis_name="c", subcore_axis_name="s")

    @functools.partial(
        pl.kernel,
        mesh=mesh,
        out_type=jax.ShapeDtypeStruct((Bn, Dt), jnp.float32),
        scratch_types=[
            pltpu.VMEM((chunk,), jnp.int32),
            pltpu.VMEM((chunk, Dt), jnp.float32),
            pltpu.SemaphoreType.DMA,
        ],
    )
    def k(table_hbm, idx_hbm, out_hbm, idx_v, rows_v, sem):
        c = lax.axis_index("c")
        s = lax.axis_index("s")
        wbase = jnp.where(c == 0, s * w0, NS * w0 + s * w1)
        nchunks = jnp.where(c == 0, n0, n1)

        def body(j, _):
            base = wbase + j * chunk
            pltpu.sync_copy(idx_hbm.at[pl.ds(base, chunk)], idx_v)
            pltpu.async_copy(table_hbm.at[idx_v], rows_v, sem).wait()
            pltpu.sync_copy(rows_v, out_hbm.at[pl.ds(base, chunk)])
            return 0

        lax.fori_loop(0, nchunks, body, 0)

    return k(table, idx)


def _sc_gather(table, idx, chunk):
    Bn = idx.shape[0]
    Dt = table.shape[1]
    info = plsc.get_sparse_core_info()
    NW = info.num_cores * info.num_subcores
    per_w = Bn // NW
    nc = per_w // chunk
    mesh = plsc.VectorSubcoreMesh(core_axis_name="c", subcore_axis_name="s")

    @functools.partial(
        pl.kernel,
        mesh=mesh,
        out_type=jax.ShapeDtypeStruct((Bn, Dt), jnp.float32),
        scratch_types=[
            pltpu.VMEM((per_w,), jnp.int32),
            pltpu.VMEM((chunk, Dt), jnp.float32),
            pltpu.VMEM((chunk, Dt), jnp.float32),
            pltpu.SemaphoreType.DMA,
            pltpu.SemaphoreType.DMA,
            pltpu.SemaphoreType.DMA,
            pltpu.SemaphoreType.DMA,
        ],
    )
    def k(table_hbm, idx_hbm, out_hbm, idx_all, buf0, buf1,
          g0, g1, s0, s1):
        wid = lax.axis_index("s") * info.num_cores + lax.axis_index("c")
        wbase = wid * per_w
        pltpu.sync_copy(idx_hbm.at[pl.ds(wbase, per_w)], idx_all)
        bufs = (buf0, buf1)
        gsems = (g0, g1)
        ssems = (s0, s1)
        gcs = [None] * nc
        sts = [None] * nc
        for j in range(nc):
            b = j & 1
            if j >= 2:
                sts[j - 2].wait()
            gcs[j] = pltpu.async_copy(
                table_hbm.at[idx_all.at[pl.ds(j * chunk, chunk)]],
                bufs[b], gsems[b])
            if j >= 1:
                gcs[j - 1].wait()
                sts[j - 1] = pltpu.async_copy(
                    bufs[(j - 1) & 1],
                    out_hbm.at[pl.ds(wbase + (j - 1) * chunk, chunk)],
                    ssems[(j - 1) & 1])
        gcs[nc - 1].wait()
        sts[nc - 1] = pltpu.async_copy(
            bufs[(nc - 1) & 1],
            out_hbm.at[pl.ds(wbase + (nc - 1) * chunk, chunk)],
            ssems[(nc - 1) & 1])
        if nc >= 2:
            sts[nc - 2].wait()
        sts[nc - 1].wait()

    return k(table, idx)


def _t0_body(g_ref, q_ref, out_ref):
    out_ref[...] = lax.dot_general(g_ref[...], q_ref[...], (((1,), (1,)), ((), ())),
                                   preferred_element_type=jnp.float32)


def _t0_sims(g, q_pad):
    return pl.pallas_call(
        _t0_body,
        grid=(N // MCH,),
        in_specs=[
            pl.BlockSpec((MCH, D), lambda n: (n, 0)),
            pl.BlockSpec((2 * B, D), lambda n: (0, 0)),
        ],
        out_specs=pl.BlockSpec((MCH, 2 * B), lambda n: (n, 0)),
        out_shape=jax.ShapeDtypeStruct((N, 2 * B), jnp.float32),
    )(g, q_pad)


T1TILE = 128
NT1 = GW // T1TILE
CP = C + 1


def _t1_body(starts_ref, ends_ref, sims_hbm, att_ref, pos_ref,
             win0_ref, win1_ref, sem0, sem1):
    p = pl.program_id(0)
    s0 = starts_ref[2 * p]
    e0 = ends_ref[2 * p]
    s1 = starts_ref[2 * p + 1]
    e1 = ends_ref[2 * p + 1]
    base0 = jnp.minimum((s0 // 8) * 8, NPAD - GW)
    base1 = jnp.minimum((s1 // 8) * 8, NPAD - GW)
    cp0 = pltpu.make_async_copy(sims_hbm.at[pl.ds(base0, GW)], win0_ref, sem0)
    cp1 = pltpu.make_async_copy(sims_hbm.at[pl.ds(base1, GW)], win1_ref, sem1)
    cp0.start()
    cp1.start()
    cp0.wait()
    cp1.wait()
    lane = lax.broadcasted_iota(jnp.int32, (1, 2 * B), 1)
    left = lane < B
    lo_v = jnp.where(left, s0 - base0, s1 - base1)
    hi_v = jnp.where(left, e0 - base0, e1 - base1)
    base_v = jnp.where(left, base0, base1)
    tio = lax.broadcasted_iota(jnp.int32, (T1TILE, 2 * B), 0)
    lmask = lax.broadcasted_iota(jnp.int32, (T1TILE, 2 * B), 1) < B
    cv = []
    ci = []
    for t in range(NT1):
        St0 = win0_ref[pl.ds(t * T1TILE, T1TILE), :]
        St1 = win1_ref[pl.ds(t * T1TILE, T1TILE), :]
        St = jnp.where(lmask, St0, St1)
        rio = tio + t * T1TILE
        cur = jnp.where((rio >= lo_v) & (rio < hi_v), St, NEG)
        for _ in range(R):
            m = jnp.max(cur, axis=0, keepdims=True)
            idx = jnp.min(jnp.where(cur == m, rio, BIGI), axis=0, keepdims=True)
            cv.append(m)
            ci.append(idx)
            cur = jnp.where(rio == idx, NEG, cur)
    V = jnp.concatenate(cv, axis=0)
    I = jnp.concatenate(ci, axis=0)
    cio = lax.broadcasted_iota(jnp.int32, (NT1 * R, 2 * B), 0)
    vals = []
    poss = []
    for _ in range(R):
        m = jnp.max(V, axis=0, keepdims=True)
        pick = jnp.min(jnp.where(V == m, cio, BIGI), axis=0, keepdims=True)
        sel = jnp.sum(jnp.where(cio == pick, I, 0), axis=0, keepdims=True)
        vals.append(m)
        poss.append(sel + base_v)
        V = jnp.where(cio == pick, NEG, V)
    v8 = jnp.concatenate(vals, axis=0)
    mw = jnp.max(v8, axis=0, keepdims=True)
    ew = jnp.exp(v8 - mw)
    att = ew / jnp.sum(ew, axis=0, keepdims=True)
    pos = jnp.concatenate(poss, axis=0)
    att_ref[...] = jnp.concatenate(
        [att[:, :B][None], att[:, B:][None]], axis=0)
    pos_ref[...] = jnp.concatenate(
        [pos[:, :B][None], pos[:, B:][None]], axis=0)


def _t1_topk(starts, ends, sims_sorted):
    return pl.pallas_call(
        _t1_body,
        grid=(CP // 2,),
        in_specs=[
            pl.BlockSpec(memory_space=pltpu.MemorySpace.SMEM),
            pl.BlockSpec(memory_space=pltpu.MemorySpace.SMEM),
            pl.BlockSpec(memory_space=pltpu.MemorySpace.HBM),
        ],
        out_specs=[
            pl.BlockSpec((2, R, B), lambda p: (p, 0, 0)),
            pl.BlockSpec((2, R, B), lambda p: (p, 0, 0)),
        ],
        out_shape=[
            jax.ShapeDtypeStruct((CP, R, B), jnp.float32),
            jax.ShapeDtypeStruct((CP, R, B), jnp.int32),
        ],
        scratch_shapes=[
            pltpu.VMEM((GW, 2 * B), jnp.float32),
            pltpu.VMEM((GW, 2 * B), jnp.float32),
            pltpu.SemaphoreType.DMA,
            pltpu.SemaphoreType.DMA,
        ],
    )(starts, ends, sims_sorted)


def _t2_body(q_ref, x_ref, att_ref, kstarts_ref, kends_ref, khbm_ref,
             out_img_ref, out_know_ref, kseg0_ref, kseg1_ref, sem0, sem1):
    c = pl.program_id(0)
    ks = kstarts_ref[c]
    ke = kends_ref[c]
    base = jnp.minimum((ks // 8) * 8, KPAD - KWIN)

    @pl.when(c == 0)
    def _prime():
        pltpu.make_async_copy(khbm_ref.at[pl.ds(base, KWIN)], kseg0_ref,
                              sem0).start()

    nxt = jnp.minimum(c + 1, C - 1)
    nks = kstarts_ref[nxt]
    nbase = jnp.minimum((nks // 8) * 8, KPAD - KWIN)

    @pl.when((c + 1 < C) & (c % 2 == 0))
    def _pf_odd():
        pltpu.make_async_copy(khbm_ref.at[pl.ds(nbase, KWIN)], kseg1_ref,
                              sem1).start()

    @pl.when((c + 1 < C) & (c % 2 == 1))
    def _pf_even():
        pltpu.make_async_copy(khbm_ref.at[pl.ds(nbase, KWIN)], kseg0_ref,
                              sem0).start()

    @pl.when(c % 2 == 0)
    def _wait_even():
        pltpu.make_async_copy(khbm_ref.at[pl.ds(0, KWIN)], kseg0_ref,
                              sem0).wait()

    @pl.when(c % 2 == 1)
    def _wait_odd():
        pltpu.make_async_copy(khbm_ref.at[pl.ds(0, KWIN)], kseg1_ref,
                              sem1).wait()

    X = x_ref[0]
    par = (c % 2 == 0)
    Kseg = jnp.where(par, kseg0_ref[...], kseg1_ref[...])
    S2 = lax.dot_general(X, Kseg, (((1,), (1,)), ((), ())),
                         preferred_element_type=jnp.float32)
    T2T = 64
    cio = lax.broadcasted_iota(jnp.int32, (T2T, KWIN), 1) + base
    m1s = []
    t4s = []
    for t in range((R * B) // T2T):
        St = S2[t * T2T:(t + 1) * T2T, :]
        cur = jnp.where((cio >= ks) & (cio < ke), St, NEG)
        m = None
        for r in range(KR):
            m = jnp.max(cur, axis=1, keepdims=True)
            if r == 0:
                m1s.append(m)
            if r < KR - 1:
                cur = jnp.where(cur == m, NEG, cur)
        t4s.append(m)
    m1 = jnp.concatenate(m1s, axis=0)
    t4 = jnp.concatenate(t4s, axis=0)
    iota = lax.broadcasted_iota(jnp.int32, (R * B, KWIN), 1)
    colk = iota + base
    Sm = jnp.where((colk >= ks) & (colk < ke), S2, NEG)
    Wk = jnp.where(Sm >= t4, jnp.exp(Sm - m1), 0.0)
    denom = jnp.sum(Wk, axis=1, keepdims=True)
    A = Wk / denom
    per_img = lax.dot_general(A.astype(jnp.bfloat16), Kseg.astype(jnp.bfloat16),
                              (((1,), (0,)), ((), ())),
                              preferred_element_type=jnp.float32)
    att = att_ref[0]
    att_flat = jnp.concatenate([att[r:r + 1, :] for r in range(R)], axis=1)
    biota = lax.broadcasted_iota(jnp.int32, (B, R * B), 0)
    colmod = lax.broadcasted_iota(jnp.int32, (B, R * B), 1) % B
    W3 = jnp.where(colmod == biota, att_flat, 0.0)
    ctx_img = lax.dot_general(W3, X, (((1,), (0,)), ((), ())),
                              preferred_element_type=jnp.float32)
    ctx_know = lax.dot_general(W3, per_img, (((1,), (0,)), ((), ())),
                               preferred_element_type=jnp.float32)
    q = q_ref[...]
    out_img_ref[...] = (0.5 * q + 0.5 * ctx_img)[None]
    out_know_ref[...] = (0.5 * q + 0.5 * ctx_know)[None]


def _t2_stage2(q, x_img, att, kstarts, kends, k_sorted):
    return pl.pallas_call(
        _t2_body,
        grid=(C,),
        in_specs=[
            pl.BlockSpec((B, D), lambda c: (0, 0)),
            pl.BlockSpec((1, R * B, D), lambda c: (c, 0, 0)),
            pl.BlockSpec((1, R, B), lambda c: (c, 0, 0)),
            pl.BlockSpec(memory_space=pltpu.MemorySpace.SMEM),
            pl.BlockSpec(memory_space=pltpu.MemorySpace.SMEM),
            pl.BlockSpec(memory_space=pltpu.MemorySpace.HBM),
        ],
        out_specs=[
            pl.BlockSpec((1, B, D), lambda c: (c, 0, 0)),
            pl.BlockSpec((1, B, D), lambda c: (c, 0, 0)),
        ],
        out_shape=[
            jax.ShapeDtypeStruct((C, B, D), jnp.float32),
            jax.ShapeDtypeStruct((C, B, D), jnp.float32),
        ],
        scratch_shapes=[
            pltpu.VMEM((KWIN, D), jnp.float32),
            pltpu.VMEM((KWIN, D), jnp.float32),
            pltpu.SemaphoreType.DMA,
            pltpu.SemaphoreType.DMA,
        ],
    )(q, x_img, att, kstarts, kends, k_sorted)


def kernel(query_embeddings, all_image_embeddings, all_knowledge_embeddings,
           image_labels, knowledge_labels):
    classes = jnp.arange(C, dtype=jnp.int32)
    il = image_labels.astype(jnp.int32)
    kltyp = knowledge_labels.astype(jnp.int32)

    ikeys = il * 131072 + jnp.arange(N, dtype=jnp.int32)
    img_order = lax.sort(ikeys) & 131071
    starts = jnp.sum(il[None, :] < classes[:, None], axis=1).astype(jnp.int32)
    ends = jnp.sum(il[None, :] <= classes[:, None], axis=1).astype(jnp.int32)
    img_order_p = jnp.concatenate([img_order, jnp.zeros((NPAD - N,), jnp.int32)])

    kkeys = kltyp * 8192 + jnp.arange(KPOOL, dtype=jnp.int32)
    korder = lax.sort(kkeys) & 8191
    kstarts = jnp.sum(kltyp[None, :] < classes[:, None], axis=1).astype(jnp.int32)
    kends = jnp.sum(kltyp[None, :] <= classes[:, None], axis=1).astype(jnp.int32)
    korder_p = jnp.concatenate([korder, jnp.zeros((KPAD - KPOOL,), jnp.int32)])

    k_sorted = _sc_gather(all_knowledge_embeddings, korder_p, 80)
    q_pad = jnp.concatenate([query_embeddings, query_embeddings], axis=0)
    sims = _t0_sims(all_image_embeddings, q_pad)
    sims_sorted = _sc_gather_simple(sims, img_order_p, 80, 21)
    starts_p = jnp.concatenate([starts, jnp.full((CP - C,), N, jnp.int32)])
    ends_p = jnp.concatenate([ends, jnp.full((CP - C,), N, jnp.int32)])
    att, pos = _t1_topk(starts_p, ends_p, sims_sorted)
    x_img = _sc_gather_translate(all_image_embeddings, img_order_p,
                                 pos[:C].reshape(-1)).reshape(C, R * B, D)
    out_img, out_know = _t2_stage2(query_embeddings, x_img, att,
                                   kstarts, kends, k_sorted)
    return (jnp.transpose(out_img, (1, 0, 2)),
            jnp.transpose(out_know, (1, 0, 2)))


# speedup vs baseline: 1.3591x; 1.0392x over previous
import functools

import jax
import jax.numpy as jnp
from jax import lax
from jax.experimental import pallas as pl
from jax.experimental.pallas import tpu as pltpu
from jax.experimental.pallas import tpu_sc as plsc

B, N, KPOOL, D, C, R, KR = 64, 50000, 5000, 512, 25, 8, 4
NEG = -1e9
NPAD = 51200
KPAD = 5120
GW = 2560
KWIN = 384
MCH = 2000
BIGI = 2**30


def _sc_gather_translate(table, order, pos):
    Bn = pos.shape[0]
    Dt = table.shape[1]
    On = order.shape[0]
    chunk = 80
    info = plsc.get_sparse_core_info()
    NW = info.num_cores * info.num_subcores
    per_w = Bn // NW
    nchunks = per_w // chunk
    mesh = plsc.VectorSubcoreMesh(core_axis_name="c", subcore_axis_name="s")

    @functools.partial(
        pl.kernel,
        mesh=mesh,
        out_type=jax.ShapeDtypeStruct((Bn, Dt), jnp.float32),
        compiler_params=pltpu.CompilerParams(needs_layout_passes=False),
        scratch_types=[
            pltpu.VMEM((On,), jnp.int32),
            pltpu.VMEM((chunk,), jnp.int32),
            pltpu.VMEM((chunk,), jnp.int32),
            pltpu.VMEM((chunk, Dt), jnp.float32),
            pltpu.SemaphoreType.DMA,
        ],
    )
    def k(table_hbm, order_hbm, pos_hbm, out_hbm, order_v, pos_v, idx_v,
          rows_v, sem):
        wid = lax.axis_index("s") * info.num_cores + lax.axis_index("c")
        pltpu.sync_copy(order_hbm, order_v)

        def body(j, _):
            base = wid * per_w + j * chunk
            pltpu.sync_copy(pos_hbm.at[pl.ds(base, chunk)], pos_v)
            for g in range(chunk // 16):
                pv = pos_v[pl.ds(g * 16, 16)]
                idx_v[pl.ds(g * 16, 16)] = plsc.load_gather(order_v, [pv])
            pltpu.async_copy(table_hbm.at[idx_v], rows_v, sem).wait()
            pltpu.sync_copy(rows_v, out_hbm.at[pl.ds(base, chunk)])
            return 0

        lax.fori_loop(0, nchunks, body, 0)

    return k(table, order, pos)


def _sc_gather_simple(table, idx, chunk, c0_frac_32nds=16):
    Bn = idx.shape[0]
    Dt = table.shape[1]
    info = plsc.get_sparse_core_info()
    NS = info.num_subcores
    w0 = (Bn * c0_frac_32nds // 32) // (NS * chunk) * chunk
    w1 = (Bn - w0 * NS) // NS
    assert w1 % chunk == 0 and (w0 + w1) * NS == Bn
    n0 = w0 // chunk
    n1 = w1 // chunk
    mesh = plsc.VectorSubcoreMesh(core_axis_name="c", subcore_axis_name="s")

    @functools.partial(
        pl.kernel,
        mesh=mesh,
        out_type=jax.ShapeDtypeStruct((Bn, Dt), jnp.float32),
        scratch_types=[
            pltpu.VMEM((chunk,), jnp.int32),
            pltpu.VMEM((chunk, Dt), jnp.float32),
            pltpu.SemaphoreType.DMA,
        ],
    )
    def k(table_hbm, idx_hbm, out_hbm, idx_v, rows_v, sem):
        c = lax.axis_index("c")
        s = lax.axis_index("s")
        wbase = jnp.where(c == 0, s * w0, NS * w0 + s * w1)
        nchunks = jnp.where(c == 0, n0, n1)

        def body(j, _):
            base = wbase + j * chunk
            pltpu.sync_copy(idx_hbm.at[pl.ds(base, chunk)], idx_v)
            pltpu.async_copy(table_hbm.at[idx_v], rows_v, sem).wait()
            pltpu.sync_copy(rows_v, out_hbm.at[pl.ds(base, chunk)])
            return 0

        lax.fori_loop(0, nchunks, body, 0)

    return k(table, idx)


def _sc_gather(table, idx, chunk):
    Bn = idx.shape[0]
    Dt = table.shape[1]
    info = plsc.get_sparse_core_info()
    NW = info.num_cores * info.num_subcores
    per_w = Bn // NW
    nc = per_w // chunk
    mesh = plsc.VectorSubcoreMesh(core_axis_name="c", subcore_axis_name="s")

    @functools.partial(
        pl.kernel,
        mesh=mesh,
        out_type=jax.ShapeDtypeStruct((Bn, Dt), jnp.float32),
        scratch_types=[
            pltpu.VMEM((per_w,), jnp.int32),
            pltpu.VMEM((chunk, Dt), jnp.float32),
            pltpu.VMEM((chunk, Dt), jnp.float32),
            pltpu.SemaphoreType.DMA,
            pltpu.SemaphoreType.DMA,
            pltpu.SemaphoreType.DMA,
            pltpu.SemaphoreType.DMA,
        ],
    )
    def k(table_hbm, idx_hbm, out_hbm, idx_all, buf0, buf1,
          g0, g1, s0, s1):
        wid = lax.axis_index("s") * info.num_cores + lax.axis_index("c")
        wbase = wid * per_w
        pltpu.sync_copy(idx_hbm.at[pl.ds(wbase, per_w)], idx_all)
        bufs = (buf0, buf1)
        gsems = (g0, g1)
        ssems = (s0, s1)
        gcs = [None] * nc
        sts = [None] * nc
        for j in range(nc):
            b = j & 1
            if j >= 2:
                sts[j - 2].wait()
            gcs[j] = pltpu.async_copy(
                table_hbm.at[idx_all.at[pl.ds(j * chunk, chunk)]],
                bufs[b], gsems[b])
            if j >= 1:
                gcs[j - 1].wait()
                sts[j - 1] = pltpu.async_copy(
                    bufs[(j - 1) & 1],
                    out_hbm.at[pl.ds(wbase + (j - 1) * chunk, chunk)],
                    ssems[(j - 1) & 1])
        gcs[nc - 1].wait()
        sts[nc - 1] = pltpu.async_copy(
            bufs[(nc - 1) & 1],
            out_hbm.at[pl.ds(wbase + (nc - 1) * chunk, chunk)],
            ssems[(nc - 1) & 1])
        if nc >= 2:
            sts[nc - 2].wait()
        sts[nc - 1].wait()

    return k(table, idx)


def _t0_body(g_ref, q_ref, out_ref):
    out_ref[...] = lax.dot_general(g_ref[...], q_ref[...], (((1,), (1,)), ((), ())),
                                   preferred_element_type=jnp.float32)


def _t0_sims(g, q_pad):
    return pl.pallas_call(
        _t0_body,
        grid=(N // MCH,),
        in_specs=[
            pl.BlockSpec((MCH, D), lambda n: (n, 0)),
            pl.BlockSpec((2 * B, D), lambda n: (0, 0)),
        ],
        out_specs=pl.BlockSpec((MCH, 2 * B), lambda n: (n, 0)),
        out_shape=jax.ShapeDtypeStruct((N, 2 * B), jnp.float32),
    )(g, q_pad)


T1TILE = 128
NT1 = GW // T1TILE
CP = C + 1


def _t1_body(starts_ref, ends_ref, sims_hbm, att_ref, pos_ref,
             we0_ref, we1_ref, wo0_ref, wo1_ref, se0, se1, so0, so1):
    p = pl.program_id(0)
    NP2 = CP // 2

    def _bases(step):
        sa = starts_ref[2 * step]
        sb = starts_ref[2 * step + 1]
        return (jnp.minimum((sa // 8) * 8, NPAD - GW),
                jnp.minimum((sb // 8) * 8, NPAD - GW))

    base0, base1 = _bases(p)

    @pl.when(p == 0)
    def _prime():
        pltpu.make_async_copy(sims_hbm.at[pl.ds(base0, GW)], we0_ref, se0).start()
        pltpu.make_async_copy(sims_hbm.at[pl.ds(base1, GW)], we1_ref, se1).start()

    nb0, nb1 = _bases(jnp.minimum(p + 1, NP2 - 1))

    @pl.when((p + 1 < NP2) & (p % 2 == 0))
    def _pf_odd():
        pltpu.make_async_copy(sims_hbm.at[pl.ds(nb0, GW)], wo0_ref, so0).start()
        pltpu.make_async_copy(sims_hbm.at[pl.ds(nb1, GW)], wo1_ref, so1).start()

    @pl.when((p + 1 < NP2) & (p % 2 == 1))
    def _pf_even():
        pltpu.make_async_copy(sims_hbm.at[pl.ds(nb0, GW)], we0_ref, se0).start()
        pltpu.make_async_copy(sims_hbm.at[pl.ds(nb1, GW)], we1_ref, se1).start()

    @pl.when(p % 2 == 0)
    def _wait_even():
        pltpu.make_async_copy(sims_hbm.at[pl.ds(0, GW)], we0_ref, se0).wait()
        pltpu.make_async_copy(sims_hbm.at[pl.ds(0, GW)], we1_ref, se1).wait()

    @pl.when(p % 2 == 1)
    def _wait_odd():
        pltpu.make_async_copy(sims_hbm.at[pl.ds(0, GW)], wo0_ref, so0).wait()
        pltpu.make_async_copy(sims_hbm.at[pl.ds(0, GW)], wo1_ref, so1).wait()

    s0 = starts_ref[2 * p]
    e0 = ends_ref[2 * p]
    s1 = starts_ref[2 * p + 1]
    e1 = ends_ref[2 * p + 1]
    par = (p % 2 == 0)
    lane = lax.broadcasted_iota(jnp.int32, (1, 2 * B), 1)
    left = lane < B
    lo_v = jnp.where(left, s0 - base0, s1 - base1)
    hi_v = jnp.where(left, e0 - base0, e1 - base1)
    base_v = jnp.where(left, base0, base1)
    tio = lax.broadcasted_iota(jnp.int32, (T1TILE, 2 * B), 0)
    lmask = lax.broadcasted_iota(jnp.int32, (T1TILE, 2 * B), 1) < B
    cv = []
    ci = []
    for t in range(NT1):
        Se0 = we0_ref[pl.ds(t * T1TILE, T1TILE), :]
        Se1 = we1_ref[pl.ds(t * T1TILE, T1TILE), :]
        So0 = wo0_ref[pl.ds(t * T1TILE, T1TILE), :]
        So1 = wo1_ref[pl.ds(t * T1TILE, T1TILE), :]
        St0 = jnp.where(par, Se0, So0)
        St1 = jnp.where(par, Se1, So1)
        St = jnp.where(lmask, St0, St1)
        rio = tio + t * T1TILE
        cur = jnp.where((rio >= lo_v) & (rio < hi_v), St, NEG)
        for _ in range(R):
            m = jnp.max(cur, axis=0, keepdims=True)
            idx = jnp.min(jnp.where(cur == m, rio, BIGI), axis=0, keepdims=True)
            cv.append(m)
            ci.append(idx)
            cur = jnp.where(rio == idx, NEG, cur)
    V = jnp.concatenate(cv, axis=0)
    I = jnp.concatenate(ci, axis=0)
    cio = lax.broadcasted_iota(jnp.int32, (NT1 * R, 2 * B), 0)
    vals = []
    poss = []
    for _ in range(R):
        m = jnp.max(V, axis=0, keepdims=True)
        pick = jnp.min(jnp.where(V == m, cio, BIGI), axis=0, keepdims=True)
        sel = jnp.sum(jnp.where(cio == pick, I, 0), axis=0, keepdims=True)
        vals.append(m)
        poss.append(sel + base_v)
        V = jnp.where(cio == pick, NEG, V)
    v8 = jnp.concatenate(vals, axis=0)
    mw = jnp.max(v8, axis=0, keepdims=True)
    ew = jnp.exp(v8 - mw)
    att = ew / jnp.sum(ew, axis=0, keepdims=True)
    pos = jnp.concatenate(poss, axis=0)
    att_ref[...] = jnp.concatenate(
        [att[:, :B][None], att[:, B:][None]], axis=0)
    pos_ref[...] = jnp.concatenate(
        [pos[:, :B][None], pos[:, B:][None]], axis=0)


def _t1_topk(starts, ends, sims_sorted):
    return pl.pallas_call(
        _t1_body,
        grid=(CP // 2,),
        in_specs=[
            pl.BlockSpec(memory_space=pltpu.MemorySpace.SMEM),
            pl.BlockSpec(memory_space=pltpu.MemorySpace.SMEM),
            pl.BlockSpec(memory_space=pltpu.MemorySpace.HBM),
        ],
        out_specs=[
            pl.BlockSpec((2, R, B), lambda p: (p, 0, 0)),
            pl.BlockSpec((2, R, B), lambda p: (p, 0, 0)),
        ],
        out_shape=[
            jax.ShapeDtypeStruct((CP, R, B), jnp.float32),
            jax.ShapeDtypeStruct((CP, R, B), jnp.int32),
        ],
        scratch_shapes=[
            pltpu.VMEM((GW, 2 * B), jnp.float32),
            pltpu.VMEM((GW, 2 * B), jnp.float32),
            pltpu.VMEM((GW, 2 * B), jnp.float32),
            pltpu.VMEM((GW, 2 * B), jnp.float32),
            pltpu.SemaphoreType.DMA,
            pltpu.SemaphoreType.DMA,
            pltpu.SemaphoreType.DMA,
            pltpu.SemaphoreType.DMA,
        ],
    )(starts, ends, sims_sorted)


def _t2_body(q_ref, x_ref, att_ref, kstarts_ref, kends_ref, khbm_ref,
             out_img_ref, out_know_ref, kseg0_ref, kseg1_ref, sem0, sem1):
    c = pl.program_id(0)
    ks = kstarts_ref[c]
    ke = kends_ref[c]
    base = jnp.minimum((ks // 8) * 8, KPAD - KWIN)

    @pl.when(c == 0)
    def _prime():
        pltpu.make_async_copy(khbm_ref.at[pl.ds(base, KWIN)], kseg0_ref,
                              sem0).start()

    nxt = jnp.minimum(c + 1, C - 1)
    nks = kstarts_ref[nxt]
    nbase = jnp.minimum((nks // 8) * 8, KPAD - KWIN)

    @pl.when((c + 1 < C) & (c % 2 == 0))
    def _pf_odd():
        pltpu.make_async_copy(khbm_ref.at[pl.ds(nbase, KWIN)], kseg1_ref,
                              sem1).start()

    @pl.when((c + 1 < C) & (c % 2 == 1))
    def _pf_even():
        pltpu.make_async_copy(khbm_ref.at[pl.ds(nbase, KWIN)], kseg0_ref,
                              sem0).start()

    @pl.when(c % 2 == 0)
    def _wait_even():
        pltpu.make_async_copy(khbm_ref.at[pl.ds(0, KWIN)], kseg0_ref,
                              sem0).wait()

    @pl.when(c % 2 == 1)
    def _wait_odd():
        pltpu.make_async_copy(khbm_ref.at[pl.ds(0, KWIN)], kseg1_ref,
                              sem1).wait()

    X = x_ref[0]
    par = (c % 2 == 0)
    Kseg = jnp.where(par, kseg0_ref[...], kseg1_ref[...])
    S2 = lax.dot_general(X, Kseg, (((1,), (1,)), ((), ())),
                         preferred_element_type=jnp.float32)
    T2T = 64
    cio = lax.broadcasted_iota(jnp.int32, (T2T, KWIN), 1) + base
    m1s = []
    t4s = []
    for t in range((R * B) // T2T):
        St = S2[t * T2T:(t + 1) * T2T, :]
        cur = jnp.where((cio >= ks) & (cio < ke), St, NEG)
        m = None
        for r in range(KR):
            m = jnp.max(cur, axis=1, keepdims=True)
            if r == 0:
                m1s.append(m)
            if r < KR - 1:
                cur = jnp.where(cur == m, NEG, cur)
        t4s.append(m)
    m1 = jnp.concatenate(m1s, axis=0)
    t4 = jnp.concatenate(t4s, axis=0)
    iota = lax.broadcasted_iota(jnp.int32, (R * B, KWIN), 1)
    colk = iota + base
    Sm = jnp.where((colk >= ks) & (colk < ke), S2, NEG)
    Wk = jnp.where(Sm >= t4, jnp.exp(Sm - m1), 0.0)
    denom = jnp.sum(Wk, axis=1, keepdims=True)
    A = Wk / denom
    per_img = lax.dot_general(A.astype(jnp.bfloat16), Kseg.astype(jnp.bfloat16),
                              (((1,), (0,)), ((), ())),
                              preferred_element_type=jnp.float32)
    att = att_ref[0]
    att_flat = jnp.concatenate([att[r:r + 1, :] for r in range(R)], axis=1)
    biota = lax.broadcasted_iota(jnp.int32, (B, R * B), 0)
    colmod = lax.broadcasted_iota(jnp.int32, (B, R * B), 1) % B
    W3 = jnp.where(colmod == biota, att_flat, 0.0)
    ctx_img = lax.dot_general(W3, X, (((1,), (0,)), ((), ())),
                              preferred_element_type=jnp.float32)
    ctx_know = lax.dot_general(W3, per_img, (((1,), (0,)), ((), ())),
                               preferred_element_type=jnp.float32)
    q = q_ref[...]
    out_img_ref[...] = (0.5 * q + 0.5 * ctx_img)[None]
    out_know_ref[...] = (0.5 * q + 0.5 * ctx_know)[None]


def _t2_stage2(q, x_img, att, kstarts, kends, k_sorted):
    return pl.pallas_call(
        _t2_body,
        grid=(C,),
        in_specs=[
            pl.BlockSpec((B, D), lambda c: (0, 0)),
            pl.BlockSpec((1, R * B, D), lambda c: (c, 0, 0)),
            pl.BlockSpec((1, R, B), lambda c: (c, 0, 0)),
            pl.BlockSpec(memory_space=pltpu.MemorySpace.SMEM),
            pl.BlockSpec(memory_space=pltpu.MemorySpace.SMEM),
            pl.BlockSpec(memory_space=pltpu.MemorySpace.HBM),
        ],
        out_specs=[
            pl.BlockSpec((1, B, D), lambda c: (c, 0, 0)),
            pl.BlockSpec((1, B, D), lambda c: (c, 0, 0)),
        ],
        out_shape=[
            jax.ShapeDtypeStruct((C, B, D), jnp.float32),
            jax.ShapeDtypeStruct((C, B, D), jnp.float32),
        ],
        scratch_shapes=[
            pltpu.VMEM((KWIN, D), jnp.float32),
            pltpu.VMEM((KWIN, D), jnp.float32),
            pltpu.SemaphoreType.DMA,
            pltpu.SemaphoreType.DMA,
        ],
    )(q, x_img, att, kstarts, kends, k_sorted)


def kernel(query_embeddings, all_image_embeddings, all_knowledge_embeddings,
           image_labels, knowledge_labels):
    classes = jnp.arange(C, dtype=jnp.int32)
    il = image_labels.astype(jnp.int32)
    kltyp = knowledge_labels.astype(jnp.int32)

    ikeys = il * 131072 + jnp.arange(N, dtype=jnp.int32)
    img_order = lax.sort(ikeys) & 131071
    starts = jnp.sum(il[None, :] < classes[:, None], axis=1).astype(jnp.int32)
    ends = jnp.sum(il[None, :] <= classes[:, None], axis=1).astype(jnp.int32)
    img_order_p = jnp.concatenate([img_order, jnp.zeros((NPAD - N,), jnp.int32)])

    kkeys = kltyp * 8192 + jnp.arange(KPOOL, dtype=jnp.int32)
    korder = lax.sort(kkeys) & 8191
    kstarts = jnp.sum(kltyp[None, :] < classes[:, None], axis=1).astype(jnp.int32)
    kends = jnp.sum(kltyp[None, :] <= classes[:, None], axis=1).astype(jnp.int32)
    korder_p = jnp.concatenate([korder, jnp.zeros((KPAD - KPOOL,), jnp.int32)])

    k_sorted = _sc_gather(all_knowledge_embeddings, korder_p, 80)
    q_pad = jnp.concatenate([query_embeddings, query_embeddings], axis=0)
    sims = _t0_sims(all_image_embeddings, q_pad)
    sims_sorted = _sc_gather_simple(sims, img_order_p, 80, 21)
    starts_p = jnp.concatenate([starts, jnp.full((CP - C,), N, jnp.int32)])
    ends_p = jnp.concatenate([ends, jnp.full((CP - C,), N, jnp.int32)])
    att, pos = _t1_topk(starts_p, ends_p, sims_sorted)
    x_img = _sc_gather_translate(all_image_embeddings, img_order_p,
                                 pos[:C].reshape(-1)).reshape(C, R * B, D)
    out_img, out_know = _t2_stage2(query_embeddings, x_img, att,
                                   kstarts, kends, k_sorted)
    return (jnp.transpose(out_img, (1, 0, 2)),
            jnp.transpose(out_know, (1, 0, 2)))


# speedup vs baseline: 1.3775x; 1.0135x over previous
import functools

import jax
import jax.numpy as jnp
from jax import lax
from jax.experimental import pallas as pl
from jax.experimental.pallas import tpu as pltpu
from jax.experimental.pallas import tpu_sc as plsc

B, N, KPOOL, D, C, R, KR = 64, 50000, 5000, 512, 25, 8, 4
NEG = -1e9
NPAD = 51200
KPAD = 5120
GW = 2560
KWIN = 384
MCH = 2000
BIGI = 2**30


def _sc_gather_translate(table, order, pos):
    Bn = pos.shape[0]
    Dt = table.shape[1]
    On = order.shape[0]
    chunk = 80
    info = plsc.get_sparse_core_info()
    NW = info.num_cores * info.num_subcores
    per_w = Bn // NW
    nchunks = per_w // chunk
    mesh = plsc.VectorSubcoreMesh(core_axis_name="c", subcore_axis_name="s")

    @functools.partial(
        pl.kernel,
        mesh=mesh,
        out_type=jax.ShapeDtypeStruct((Bn, Dt), jnp.float32),
        compiler_params=pltpu.CompilerParams(needs_layout_passes=False),
        scratch_types=[
            pltpu.VMEM((On,), jnp.int32),
            pltpu.VMEM((chunk,), jnp.int32),
            pltpu.VMEM((chunk,), jnp.int32),
            pltpu.VMEM((chunk, Dt), jnp.float32),
            pltpu.SemaphoreType.DMA,
        ],
    )
    def k(table_hbm, order_hbm, pos_hbm, out_hbm, order_v, pos_v, idx_v,
          rows_v, sem):
        wid = lax.axis_index("s") * info.num_cores + lax.axis_index("c")
        pltpu.sync_copy(order_hbm, order_v)

        def body(j, _):
            base = wid * per_w + j * chunk
            pltpu.sync_copy(pos_hbm.at[pl.ds(base, chunk)], pos_v)
            for g in range(chunk // 16):
                pv = pos_v[pl.ds(g * 16, 16)]
                idx_v[pl.ds(g * 16, 16)] = plsc.load_gather(order_v, [pv])
            pltpu.async_copy(table_hbm.at[idx_v], rows_v, sem).wait()
            pltpu.sync_copy(rows_v, out_hbm.at[pl.ds(base, chunk)])
            return 0

        lax.fori_loop(0, nchunks, body, 0)

    return k(table, order, pos)


def _sc_gather_simple(table, idx, chunk, c0_frac_32nds=16):
    Bn = idx.shape[0]
    Dt = table.shape[1]
    info = plsc.get_sparse_core_info()
    NS = info.num_subcores
    w0 = (Bn * c0_frac_32nds // 32) // (NS * chunk) * chunk
    w1 = (Bn - w0 * NS) // NS
    assert w1 % chunk == 0 and (w0 + w1) * NS == Bn
    n0 = w0 // chunk
    n1 = w1 // chunk
    mesh = plsc.VectorSubcoreMesh(core_axis_name="c", subcore_axis_name="s")

    wmax = max(w0, w1)

    @functools.partial(
        pl.kernel,
        mesh=mesh,
        out_type=jax.ShapeDtypeStruct((Bn, Dt), jnp.float32),
        scratch_types=[
            pltpu.VMEM((wmax,), jnp.int32),
            pltpu.VMEM((chunk, Dt), jnp.float32),
            pltpu.SemaphoreType.DMA,
        ],
    )
    def k(table_hbm, idx_hbm, out_hbm, idx_all, rows_v, sem):
        c = lax.axis_index("c")
        s = lax.axis_index("s")
        wbase = jnp.where(c == 0, s * w0, NS * w0 + s * w1)
        nchunks = jnp.where(c == 0, n0, n1)
        pltpu.async_copy(idx_hbm.at[pl.ds(wbase, wmax)], idx_all, sem).wait()

        def body(j, _):
            base = wbase + j * chunk
            pltpu.async_copy(
                table_hbm.at[idx_all.at[pl.ds(j * chunk, chunk)]],
                rows_v, sem).wait()
            pltpu.sync_copy(rows_v, out_hbm.at[pl.ds(base, chunk)])
            return 0

        lax.fori_loop(0, nchunks, body, 0)

    return k(table, idx)


def _sc_gather(table, idx, chunk):
    Bn = idx.shape[0]
    Dt = table.shape[1]
    info = plsc.get_sparse_core_info()
    NW = info.num_cores * info.num_subcores
    per_w = Bn // NW
    nc = per_w // chunk
    mesh = plsc.VectorSubcoreMesh(core_axis_name="c", subcore_axis_name="s")

    @functools.partial(
        pl.kernel,
        mesh=mesh,
        out_type=jax.ShapeDtypeStruct((Bn, Dt), jnp.float32),
        scratch_types=[
            pltpu.VMEM((per_w,), jnp.int32),
            pltpu.VMEM((chunk, Dt), jnp.float32),
            pltpu.VMEM((chunk, Dt), jnp.float32),
            pltpu.SemaphoreType.DMA,
            pltpu.SemaphoreType.DMA,
            pltpu.SemaphoreType.DMA,
            pltpu.SemaphoreType.DMA,
        ],
    )
    def k(table_hbm, idx_hbm, out_hbm, idx_all, buf0, buf1,
          g0, g1, s0, s1):
        wid = lax.axis_index("s") * info.num_cores + lax.axis_index("c")
        wbase = wid * per_w
        pltpu.sync_copy(idx_hbm.at[pl.ds(wbase, per_w)], idx_all)
        bufs = (buf0, buf1)
        gsems = (g0, g1)
        ssems = (s0, s1)
        gcs = [None] * nc
        sts = [None] * nc
        for j in range(nc):
            b = j & 1
            if j >= 2:
                sts[j - 2].wait()
            gcs[j] = pltpu.async_copy(
                table_hbm.at[idx_all.at[pl.ds(j * chunk, chunk)]],
                bufs[b], gsems[b])
            if j >= 1:
                gcs[j - 1].wait()
                sts[j - 1] = pltpu.async_copy(
                    bufs[(j - 1) & 1],
                    out_hbm.at[pl.ds(wbase + (j - 1) * chunk, chunk)],
                    ssems[(j - 1) & 1])
        gcs[nc - 1].wait()
        sts[nc - 1] = pltpu.async_copy(
            bufs[(nc - 1) & 1],
            out_hbm.at[pl.ds(wbase + (nc - 1) * chunk, chunk)],
            ssems[(nc - 1) & 1])
        if nc >= 2:
            sts[nc - 2].wait()
        sts[nc - 1].wait()

    return k(table, idx)


def _t0_body(g_ref, q_ref, out_ref):
    out_ref[...] = lax.dot_general(g_ref[...], q_ref[...], (((1,), (1,)), ((), ())),
                                   preferred_element_type=jnp.float32)


def _t0_sims(g, q_pad):
    return pl.pallas_call(
        _t0_body,
        grid=(N // MCH,),
        in_specs=[
            pl.BlockSpec((MCH, D), lambda n: (n, 0)),
            pl.BlockSpec((2 * B, D), lambda n: (0, 0)),
        ],
        out_specs=pl.BlockSpec((MCH, 2 * B), lambda n: (n, 0)),
        out_shape=jax.ShapeDtypeStruct((N, 2 * B), jnp.float32),
    )(g, q_pad)


T1TILE = 128
NT1 = GW // T1TILE
CP = C + 1


def _t1_body(starts_ref, ends_ref, sims_hbm, att_ref, pos_ref,
             we0_ref, we1_ref, wo0_ref, wo1_ref, se0, se1, so0, so1):
    p = pl.program_id(0)
    NP2 = CP // 2

    def _bases(step):
        sa = starts_ref[2 * step]
        sb = starts_ref[2 * step + 1]
        return (jnp.minimum((sa // 8) * 8, NPAD - GW),
                jnp.minimum((sb // 8) * 8, NPAD - GW))

    base0, base1 = _bases(p)

    @pl.when(p == 0)
    def _prime():
        pltpu.make_async_copy(sims_hbm.at[pl.ds(base0, GW)], we0_ref, se0).start()
        pltpu.make_async_copy(sims_hbm.at[pl.ds(base1, GW)], we1_ref, se1).start()

    nb0, nb1 = _bases(jnp.minimum(p + 1, NP2 - 1))

    @pl.when((p + 1 < NP2) & (p % 2 == 0))
    def _pf_odd():
        pltpu.make_async_copy(sims_hbm.at[pl.ds(nb0, GW)], wo0_ref, so0).start()
        pltpu.make_async_copy(sims_hbm.at[pl.ds(nb1, GW)], wo1_ref, so1).start()

    @pl.when((p + 1 < NP2) & (p % 2 == 1))
    def _pf_even():
        pltpu.make_async_copy(sims_hbm.at[pl.ds(nb0, GW)], we0_ref, se0).start()
        pltpu.make_async_copy(sims_hbm.at[pl.ds(nb1, GW)], we1_ref, se1).start()

    @pl.when(p % 2 == 0)
    def _wait_even():
        pltpu.make_async_copy(sims_hbm.at[pl.ds(0, GW)], we0_ref, se0).wait()
        pltpu.make_async_copy(sims_hbm.at[pl.ds(0, GW)], we1_ref, se1).wait()

    @pl.when(p % 2 == 1)
    def _wait_odd():
        pltpu.make_async_copy(sims_hbm.at[pl.ds(0, GW)], wo0_ref, so0).wait()
        pltpu.make_async_copy(sims_hbm.at[pl.ds(0, GW)], wo1_ref, so1).wait()

    s0 = starts_ref[2 * p]
    e0 = ends_ref[2 * p]
    s1 = starts_ref[2 * p + 1]
    e1 = ends_ref[2 * p + 1]
    par = (p % 2 == 0)
    lane = lax.broadcasted_iota(jnp.int32, (1, 2 * B), 1)
    left = lane < B
    lo_v = jnp.where(left, s0 - base0, s1 - base1)
    hi_v = jnp.where(left, e0 - base0, e1 - base1)
    base_v = jnp.where(left, base0, base1)
    tio = lax.broadcasted_iota(jnp.int32, (T1TILE, 2 * B), 0)
    lmask = lax.broadcasted_iota(jnp.int32, (T1TILE, 2 * B), 1) < B
    cv = []
    ci = []
    for t in range(NT1):
        Se0 = we0_ref[pl.ds(t * T1TILE, T1TILE), :]
        Se1 = we1_ref[pl.ds(t * T1TILE, T1TILE), :]
        So0 = wo0_ref[pl.ds(t * T1TILE, T1TILE), :]
        So1 = wo1_ref[pl.ds(t * T1TILE, T1TILE), :]
        St0 = jnp.where(par, Se0, So0)
        St1 = jnp.where(par, Se1, So1)
        St = jnp.where(lmask, St0, St1)
        rio = tio + t * T1TILE
        cur = jnp.where((rio >= lo_v) & (rio < hi_v), St, NEG)
        for _ in range(R):
            m = jnp.max(cur, axis=0, keepdims=True)
            idx = jnp.min(jnp.where(cur == m, rio, BIGI), axis=0, keepdims=True)
            cv.append(m)
            ci.append(idx)
            cur = jnp.where(rio == idx, NEG, cur)
    V = jnp.concatenate(cv, axis=0)
    I = jnp.concatenate(ci, axis=0)
    cio = lax.broadcasted_iota(jnp.int32, (NT1 * R, 2 * B), 0)
    vals = []
    poss = []
    for _ in range(R):
        m = jnp.max(V, axis=0, keepdims=True)
        pick = jnp.min(jnp.where(V == m, cio, BIGI), axis=0, keepdims=True)
        sel = jnp.sum(jnp.where(cio == pick, I, 0), axis=0, keepdims=True)
        vals.append(m)
        poss.append(sel + base_v)
        V = jnp.where(cio == pick, NEG, V)
    v8 = jnp.concatenate(vals, axis=0)
    mw = jnp.max(v8, axis=0, keepdims=True)
    ew = jnp.exp(v8 - mw)
    att = ew / jnp.sum(ew, axis=0, keepdims=True)
    pos = jnp.concatenate(poss, axis=0)
    att_ref[...] = jnp.concatenate(
        [att[:, :B][None], att[:, B:][None]], axis=0)
    pos_ref[...] = jnp.concatenate(
        [pos[:, :B][None], pos[:, B:][None]], axis=0)


def _t1_topk(starts, ends, sims_sorted):
    return pl.pallas_call(
        _t1_body,
        grid=(CP // 2,),
        in_specs=[
            pl.BlockSpec(memory_space=pltpu.MemorySpace.SMEM),
            pl.BlockSpec(memory_space=pltpu.MemorySpace.SMEM),
            pl.BlockSpec(memory_space=pltpu.MemorySpace.HBM),
        ],
        out_specs=[
            pl.BlockSpec((2, R, B), lambda p: (p, 0, 0)),
            pl.BlockSpec((2, R, B), lambda p: (p, 0, 0)),
        ],
        out_shape=[
            jax.ShapeDtypeStruct((CP, R, B), jnp.float32),
            jax.ShapeDtypeStruct((CP, R, B), jnp.int32),
        ],
        scratch_shapes=[
            pltpu.VMEM((GW, 2 * B), jnp.float32),
            pltpu.VMEM((GW, 2 * B), jnp.float32),
            pltpu.VMEM((GW, 2 * B), jnp.float32),
            pltpu.VMEM((GW, 2 * B), jnp.float32),
            pltpu.SemaphoreType.DMA,
            pltpu.SemaphoreType.DMA,
            pltpu.SemaphoreType.DMA,
            pltpu.SemaphoreType.DMA,
        ],
    )(starts, ends, sims_sorted)


def _t2_body(q_ref, x_ref, att_ref, kstarts_ref, kends_ref, khbm_ref,
             out_img_ref, out_know_ref, kseg0_ref, kseg1_ref, sem0, sem1):
    c = pl.program_id(0)
    ks = kstarts_ref[c]
    ke = kends_ref[c]
    base = jnp.minimum((ks // 8) * 8, KPAD - KWIN)

    @pl.when(c == 0)
    def _prime():
        pltpu.make_async_copy(khbm_ref.at[pl.ds(base, KWIN)], kseg0_ref,
                              sem0).start()

    nxt = jnp.minimum(c + 1, C - 1)
    nks = kstarts_ref[nxt]
    nbase = jnp.minimum((nks // 8) * 8, KPAD - KWIN)

    @pl.when((c + 1 < C) & (c % 2 == 0))
    def _pf_odd():
        pltpu.make_async_copy(khbm_ref.at[pl.ds(nbase, KWIN)], kseg1_ref,
                              sem1).start()

    @pl.when((c + 1 < C) & (c % 2 == 1))
    def _pf_even():
        pltpu.make_async_copy(khbm_ref.at[pl.ds(nbase, KWIN)], kseg0_ref,
                              sem0).start()

    @pl.when(c % 2 == 0)
    def _wait_even():
        pltpu.make_async_copy(khbm_ref.at[pl.ds(0, KWIN)], kseg0_ref,
                              sem0).wait()

    @pl.when(c % 2 == 1)
    def _wait_odd():
        pltpu.make_async_copy(khbm_ref.at[pl.ds(0, KWIN)], kseg1_ref,
                              sem1).wait()

    X = x_ref[0]
    par = (c % 2 == 0)
    Kseg = jnp.where(par, kseg0_ref[...], kseg1_ref[...])
    S2 = lax.dot_general(X, Kseg, (((1,), (1,)), ((), ())),
                         preferred_element_type=jnp.float32)
    T2T = 64
    cio = lax.broadcasted_iota(jnp.int32, (T2T, KWIN), 1) + base
    m1s = []
    t4s = []
    for t in range((R * B) // T2T):
        St = S2[t * T2T:(t + 1) * T2T, :]
        cur = jnp.where((cio >= ks) & (cio < ke), St, NEG)
        m = None
        for r in range(KR):
            m = jnp.max(cur, axis=1, keepdims=True)
            if r == 0:
                m1s.append(m)
            if r < KR - 1:
                cur = jnp.where(cur == m, NEG, cur)
        t4s.append(m)
    m1 = jnp.concatenate(m1s, axis=0)
    t4 = jnp.concatenate(t4s, axis=0)
    iota = lax.broadcasted_iota(jnp.int32, (R * B, KWIN), 1)
    colk = iota + base
    Sm = jnp.where((colk >= ks) & (colk < ke), S2, NEG)
    Wk = jnp.where(Sm >= t4, jnp.exp(Sm - m1), 0.0)
    denom = jnp.sum(Wk, axis=1, keepdims=True)
    A = Wk / denom
    per_img = lax.dot_general(A.astype(jnp.bfloat16), Kseg.astype(jnp.bfloat16),
                              (((1,), (0,)), ((), ())),
                              preferred_element_type=jnp.float32)
    att = att_ref[0]
    att_flat = jnp.concatenate([att[r:r + 1, :] for r in range(R)], axis=1)
    biota = lax.broadcasted_iota(jnp.int32, (B, R * B), 0)
    colmod = lax.broadcasted_iota(jnp.int32, (B, R * B), 1) % B
    W3 = jnp.where(colmod == biota, att_flat, 0.0)
    ctx_img = lax.dot_general(W3, X, (((1,), (0,)), ((), ())),
                              preferred_element_type=jnp.float32)
    ctx_know = lax.dot_general(W3, per_img, (((1,), (0,)), ((), ())),
                               preferred_element_type=jnp.float32)
    q = q_ref[...]
    out_img_ref[...] = (0.5 * q + 0.5 * ctx_img)[None]
    out_know_ref[...] = (0.5 * q + 0.5 * ctx_know)[None]


def _t2_stage2(q, x_img, att, kstarts, kends, k_sorted):
    return pl.pallas_call(
        _t2_body,
        grid=(C,),
        in_specs=[
            pl.BlockSpec((B, D), lambda c: (0, 0)),
            pl.BlockSpec((1, R * B, D), lambda c: (c, 0, 0)),
            pl.BlockSpec((1, R, B), lambda c: (c, 0, 0)),
            pl.BlockSpec(memory_space=pltpu.MemorySpace.SMEM),
            pl.BlockSpec(memory_space=pltpu.MemorySpace.SMEM),
            pl.BlockSpec(memory_space=pltpu.MemorySpace.HBM),
        ],
        out_specs=[
            pl.BlockSpec((1, B, D), lambda c: (c, 0, 0)),
            pl.BlockSpec((1, B, D), lambda c: (c, 0, 0)),
        ],
        out_shape=[
            jax.ShapeDtypeStruct((C, B, D), jnp.float32),
            jax.ShapeDtypeStruct((C, B, D), jnp.float32),
        ],
        scratch_shapes=[
            pltpu.VMEM((KWIN, D), jnp.float32),
            pltpu.VMEM((KWIN, D), jnp.float32),
            pltpu.SemaphoreType.DMA,
            pltpu.SemaphoreType.DMA,
        ],
    )(q, x_img, att, kstarts, kends, k_sorted)


def kernel(query_embeddings, all_image_embeddings, all_knowledge_embeddings,
           image_labels, knowledge_labels):
    classes = jnp.arange(C, dtype=jnp.int32)
    il = image_labels.astype(jnp.int32)
    kltyp = knowledge_labels.astype(jnp.int32)

    ikeys = il * 131072 + jnp.arange(N, dtype=jnp.int32)
    img_order = lax.sort(ikeys) & 131071
    starts = jnp.sum(il[None, :] < classes[:, None], axis=1).astype(jnp.int32)
    ends = jnp.sum(il[None, :] <= classes[:, None], axis=1).astype(jnp.int32)
    img_order_p = jnp.concatenate([img_order, jnp.zeros((NPAD - N,), jnp.int32)])

    kkeys = kltyp * 8192 + jnp.arange(KPOOL, dtype=jnp.int32)
    korder = lax.sort(kkeys) & 8191
    kstarts = jnp.sum(kltyp[None, :] < classes[:, None], axis=1).astype(jnp.int32)
    kends = jnp.sum(kltyp[None, :] <= classes[:, None], axis=1).astype(jnp.int32)
    korder_p = jnp.concatenate([korder, jnp.zeros((KPAD - KPOOL,), jnp.int32)])

    k_sorted = _sc_gather(all_knowledge_embeddings, korder_p, 80)
    q_pad = jnp.concatenate([query_embeddings, query_embeddings], axis=0)
    sims = _t0_sims(all_image_embeddings, q_pad)
    sims_sorted = _sc_gather_simple(sims, img_order_p, 80, 21)
    starts_p = jnp.concatenate([starts, jnp.full((CP - C,), N, jnp.int32)])
    ends_p = jnp.concatenate([ends, jnp.full((CP - C,), N, jnp.int32)])
    att, pos = _t1_topk(starts_p, ends_p, sims_sorted)
    x_img = _sc_gather_translate(all_image_embeddings, img_order_p,
                                 pos[:C].reshape(-1)).reshape(C, R * B, D)
    out_img, out_know = _t2_stage2(query_embeddings, x_img, att,
                                   kstarts, kends, k_sorted)
    return (jnp.transpose(out_img, (1, 0, 2)),
            jnp.transpose(out_know, (1, 0, 2)))
